# baseline (speedup 1.0000x reference)
import jax
import jax.numpy as jnp
from jax import lax
from jax.experimental import pallas as pl
from jax.experimental.pallas import tpu as pltpu
from jax.experimental.pallas import tpu_sc as plsc

ROWS = 16
HIDDEN = 4096
NC = 2
NS = 16
NW = NC * NS
CHUNK = HIDDEN // NW


def _body(table_hbm, idx_hbm, out_hbm, idx_v, rows_v, sem):
    c = lax.axis_index("c")
    s = lax.axis_index("s")
    wid = s * NC + c

    pltpu.sync_copy(idx_hbm, idx_v)
    idx = idx_v[...]

    src_rows = idx * NW + wid
    dst_rows = lax.iota(jnp.int32, ROWS) * NW + wid

    pltpu.async_copy(table_hbm.at[src_rows], rows_v, sem).wait()
    pltpu.async_copy(rows_v, out_hbm.at[dst_rows], sem).wait()


def kernel(table, seq_indices):
    table_flat = table.reshape(ROWS * NW, CHUNK)
    mesh = plsc.VectorSubcoreMesh(core_axis_name="c", subcore_axis_name="s")
    out = pl.kernel(
        _body,
        mesh=mesh,
        out_type=jax.ShapeDtypeStruct((ROWS * NW, CHUNK), jnp.float32),
        scratch_types=[
            pltpu.VMEM((ROWS,), jnp.int32),
            pltpu.VMEM((ROWS, CHUNK), jnp.float32),
            pltpu.SemaphoreType.DMA,
        ],
    )(table_flat, seq_indices)
    return out.reshape(1, ROWS, HIDDEN)


# speedup vs baseline: 1.0736x; 1.0736x over previous
import jax
import jax.numpy as jnp
from jax import lax
from jax.experimental import pallas as pl
from jax.experimental.pallas import tpu as pltpu
from jax.experimental.pallas import tpu_sc as plsc

ROWS = 16
HIDDEN = 4096
NC = 2
NS = 16
NW = NC * NS
CHUNK = HIDDEN // NW


def _body(table_hbm, idx_hbm, out_hbm, idx_v, rows_v, sem):
    c = lax.axis_index("c")
    s = lax.axis_index("s")
    wid = s * NC + c
    col = wid * CHUNK

    pltpu.sync_copy(idx_hbm, idx_v)
    idx = idx_v[...]

    pltpu.async_copy(table_hbm.at[idx, pl.ds(col, CHUNK)], rows_v, sem).wait()
    pltpu.sync_copy(rows_v, out_hbm.at[:, pl.ds(col, CHUNK)])


def kernel(table, seq_indices):
    mesh = plsc.VectorSubcoreMesh(core_axis_name="c", subcore_axis_name="s")
    out = pl.kernel(
        _body,
        mesh=mesh,
        out_type=jax.ShapeDtypeStruct((ROWS, HIDDEN), jnp.float32),
        scratch_types=[
            pltpu.VMEM((ROWS,), jnp.int32),
            pltpu.VMEM((ROWS, CHUNK), jnp.float32),
            pltpu.SemaphoreType.DMA,
        ],
    )(table, seq_indices)
    return out[None]


# speedup vs baseline: 1.1801x; 1.0992x over previous
import jax
import jax.numpy as jnp
from jax import lax
from jax.experimental import pallas as pl
from jax.experimental.pallas import tpu as pltpu
from jax.experimental.pallas import tpu_sc as plsc

ROWS = 16
HIDDEN = 4096
NC = 1
NS = 16
NW = NC * NS
CHUNK = HIDDEN // NW


def _body(table_hbm, idx_hbm, out_hbm, idx_v, rows_v, sem):
    c = lax.axis_index("c")
    s = lax.axis_index("s")
    wid = s * NC + c
    col = wid * CHUNK

    pltpu.sync_copy(idx_hbm, idx_v)
    idx = idx_v[...]

    pltpu.async_copy(table_hbm.at[idx, pl.ds(col, CHUNK)], rows_v, sem).wait()
    pltpu.sync_copy(rows_v, out_hbm.at[:, pl.ds(col, CHUNK)])


def kernel(table, seq_indices):
    mesh = plsc.VectorSubcoreMesh(
        core_axis_name="c", subcore_axis_name="s", num_cores=1
    )
    out = pl.kernel(
        _body,
        mesh=mesh,
        out_type=jax.ShapeDtypeStruct((ROWS, HIDDEN), jnp.float32),
        scratch_types=[
            pltpu.VMEM((ROWS,), jnp.int32),
            pltpu.VMEM((ROWS, CHUNK), jnp.float32),
            pltpu.SemaphoreType.DMA,
        ],
    )(table, seq_indices)
    return out[None]
